# baseline (speedup 1.0000x reference)
import jax
import jax.numpy as jnp
from jax.experimental import pallas as pl
from jax.experimental.pallas import tpu as pltpu

_BLK = 512
_NBUF = 3
_OUT_ROWS = 8192
_N_COLS = 2048
_N_STEPS = _OUT_ROWS // _BLK


def _body(x_hbm, o_hbm, x1_v, x2_v, o_v, in1_sem, in2_sem, out_sem):
    def in1_dma(i, b):
        return pltpu.make_async_copy(
            x_hbm.at[pl.ds(i * _BLK, _BLK), :], x1_v.at[b], in1_sem.at[b]
        )

    def in2_dma(i, b):
        return pltpu.make_async_copy(
            x_hbm.at[pl.ds(_OUT_ROWS + i * _BLK, _BLK), :],
            x2_v.at[b],
            in2_sem.at[b],
        )

    def out_dma(i, b):
        return pltpu.make_async_copy(
            o_v.at[b], o_hbm.at[pl.ds(i * _BLK, _BLK), :], out_sem.at[b]
        )

    for k in range(_NBUF):
        in1_dma(k, k).start()
        in2_dma(k, k).start()

    for i in range(_N_STEPS):
        b = i % _NBUF
        in1_dma(i, b).wait()
        in2_dma(i, b).wait()
        if i >= _NBUF:
            out_dma(i - _NBUF, b).wait()
        o_v[b] = (x1_v[b] + x2_v[b]) * 0.5
        out_dma(i, b).start()
        nxt = i + _NBUF
        if nxt < _N_STEPS:
            in1_dma(nxt, b).start()
            in2_dma(nxt, b).start()

    for i in range(_N_STEPS - _NBUF, _N_STEPS):
        out_dma(i, i % _NBUF).wait()


def kernel(x_cat):
    return pl.pallas_call(
        _body,
        in_specs=[pl.BlockSpec(memory_space=pl.ANY)],
        out_specs=pl.BlockSpec(memory_space=pl.ANY),
        out_shape=jax.ShapeDtypeStruct((_OUT_ROWS, _N_COLS), x_cat.dtype),
        scratch_shapes=[
            pltpu.VMEM((_NBUF, _BLK, _N_COLS), jnp.float32),
            pltpu.VMEM((_NBUF, _BLK, _N_COLS), jnp.float32),
            pltpu.VMEM((_NBUF, _BLK, _N_COLS), jnp.float32),
            pltpu.SemaphoreType.DMA((_NBUF,)),
            pltpu.SemaphoreType.DMA((_NBUF,)),
            pltpu.SemaphoreType.DMA((_NBUF,)),
        ],
    )(x_cat)


# speedup vs baseline: 1.0020x; 1.0020x over previous
import jax
import jax.numpy as jnp
from jax.experimental import pallas as pl
from jax.experimental.pallas import tpu as pltpu

_BLK = 512


def _mean_kernel(x1_ref, x2_ref, o_ref):
    o_ref[...] = (x1_ref[...] + x2_ref[...]) * 0.5


def kernel(x_cat):
    n_rows, n_cols = x_cat.shape
    x_len = n_rows // 2
    n_blocks = x_len // _BLK
    return pl.pallas_call(
        _mean_kernel,
        grid=(n_blocks,),
        in_specs=[
            pl.BlockSpec((_BLK, n_cols), lambda i: (i, 0)),
            pl.BlockSpec(
                (_BLK, n_cols),
                lambda i, nb=n_blocks: (i + nb, 0),
            ),
        ],
        out_specs=pl.BlockSpec((_BLK, n_cols), lambda i: (i, 0)),
        out_shape=jax.ShapeDtypeStruct((x_len, n_cols), x_cat.dtype),
        compiler_params=pltpu.CompilerParams(
            dimension_semantics=("arbitrary",),
        ),
    )(x_cat, x_cat)
